# baseline (speedup 1.0000x reference)
import jax
import jax.numpy as jnp
from jax import lax
from jax.experimental import pallas as pl
from jax.experimental.pallas import tpu as pltpu
from jax.experimental.pallas import tpu_sc as plsc

N = 50000
NPAD = 50048
E = 800000
F_IN = 26
FP = 32
H = 64
G = 64
NC = 2
NS = 16
NW = NC * NS
CB = 128
NCHUNKS = E // CB
CPT = NCHUNKS // NW
NLEFT = NCHUNKS - CPT * NW
IB = 13
NBLK = CPT // IB
ROWS_PER_TILE = NPAD // NS
ZROWS = 184
R = 3128
LANES = 16


def _l1_body(xpad, src2d, dst2d, out, acc, zbuf, srcl0, dstl0, srcl1, dstl1,
             stail, dtail, msgs0, msgs1, msgs2, sem0, sem1, sem2,
             ssem0, ssem1, ssem2, isem, zsem):
    M = (msgs0, msgs1, msgs2)
    SG = (sem0, sem1, sem2)
    SS = (ssem0, ssem1, ssem2)
    c = lax.axis_index("c")
    s = lax.axis_index("s")
    w = c * NS + s

    def i_start(blk, sbuf, dbuf):
        cb0 = w * CPT + blk * IB
        pltpu.async_copy(src2d.at[pl.ds(cb0, IB), :], sbuf, isem)
        pltpu.async_copy(dst2d.at[pl.ds(cb0, IB), :], dbuf, isem)

    def i_wait(blk, sbuf, dbuf):
        cb0 = w * CPT + blk * IB
        pltpu.make_async_copy(src2d.at[pl.ds(cb0, IB), :], sbuf, isem).wait()
        pltpu.make_async_copy(dst2d.at[pl.ds(cb0, IB), :], dbuf, isem).wait()

    i_start(0, srcl0, dstl0)

    zero16 = jnp.zeros((LANES,), jnp.float32)
    for r in range(ZROWS):
        for j in range(FP // LANES):
            zbuf[r, pl.ds(j * LANES, LANES)] = zero16
    nz = ROWS_PER_TILE // ZROWS
    for i in range(nz):
        pltpu.async_copy(
            zbuf, acc.at[pl.ds(s * ROWS_PER_TILE + i * ZROWS, ZROWS), :],
            zsem)
    for i in range(nz):
        pltpu.make_async_copy(
            zbuf, acc.at[pl.ds(s * ROWS_PER_TILE + i * ZROWS, ZROWS), :],
            zsem).wait()
    plsc.subcore_barrier()

    def g_start(srcl, j, b):
        pltpu.async_copy(xpad.at[srcl.at[j]], M[b], SG[b])

    def g_wait(srcl, j, b):
        pltpu.make_async_copy(xpad.at[srcl.at[j]], M[b], SG[b]).wait()

    def s_start(dstl, j, b):
        pltpu.async_copy(M[b], acc.at[dstl.at[j]], SS[b], add=True)

    def s_wait(dstl, j, b):
        pltpu.make_async_copy(M[b], acc.at[dstl.at[j]], SS[b]).wait()

    def process(srcl, dstl):
        g_start(srcl, 0, 0)
        g_start(srcl, 1, 1)
        for ch in range(IB):
            b = ch % 3
            g_wait(srcl, ch, b)
            s_start(dstl, ch, b)
            if ch + 2 < IB:
                if ch >= 1:
                    s_wait(dstl, ch - 1, (ch - 1) % 3)
                g_start(srcl, ch + 2, (ch + 2) % 3)
        s_wait(dstl, IB - 3, (IB - 3) % 3)
        s_wait(dstl, IB - 2, (IB - 2) % 3)
        s_wait(dstl, IB - 1, (IB - 1) % 3)

    def blk_body(t, carry):
        b0 = 2 * t
        b1 = 2 * t + 1
        i_start(b1, srcl1, dstl1)
        i_wait(b0, srcl0, dstl0)
        process(srcl0, dstl0)

        @pl.when(b0 + 2 < NBLK)
        def _():
            i_start(b0 + 2, srcl0, dstl0)

        i_wait(b1, srcl1, dstl1)
        process(srcl1, dstl1)
        return carry

    lax.fori_loop(0, NBLK // 2, blk_body, 0)
    i_wait(NBLK - 1, srcl0, dstl0)
    process(srcl0, dstl0)

    @pl.when(w < NLEFT)
    def _():
        pltpu.sync_copy(src2d.at[NW * CPT + w], stail)
        pltpu.sync_copy(dst2d.at[NW * CPT + w], dtail)
        pltpu.async_copy(xpad.at[stail], msgs1, sem1).wait()
        pltpu.sync_copy(msgs1, acc.at[dtail], add=True)

    plsc.subcore_barrier()
    pltpu.sync_copy(
        acc.at[pl.ds(s * ROWS_PER_TILE, ROWS_PER_TILE), :],
        out.at[c, pl.ds(s * ROWS_PER_TILE, ROWS_PER_TILE), :])


def _scale_rows(msgs, dstall, wgtab, gbuf, base):
    for k in range(CB // LANES):
        dv = dstall[pl.ds(base + k * LANES, LANES)]
        pk = plsc.load_gather(wgtab, [dv])
        gv = pk & jnp.int32(63)
        wv = plsc.bitcast(pk & jnp.int32(-64), jnp.float32)
        gbuf[pl.ds(k * LANES, LANES)] = gv
        for e in range(LANES):
            ws = jnp.broadcast_to(wv[e], (LANES,))
            row = k * LANES + e
            for j in range(H // LANES):
                sl = pl.ds(j * LANES, LANES)
                msgs[row, sl] = msgs[row, sl] * ws


def _l2_body(h, src2d, dst1d, wg, out, acc, wgtab, srcl, dstall, stail,
             gbuf0, gbuf1, gbuf2, msgs0, msgs1, msgs2,
             sem0, sem1, sem2, ssem0, ssem1, ssem2):
    c = lax.axis_index("c")
    s = lax.axis_index("s")
    w = c * NS + s
    M = (msgs0, msgs1, msgs2)
    GB = (gbuf0, gbuf1, gbuf2)
    SG = (sem0, sem1, sem2)
    SS = (ssem0, ssem1, ssem2)

    pltpu.sync_copy(src2d.at[pl.ds(w * CPT, CPT), :], srcl)
    pltpu.sync_copy(dst1d.at[pl.ds(w * CPT * CB, CPT * CB)], dstall)
    pltpu.sync_copy(wg, wgtab)

    def g_start(j, b):
        pltpu.async_copy(h.at[srcl.at[j]], M[b], SG[b])

    def g_wait(j, b):
        pltpu.make_async_copy(h.at[srcl.at[j]], M[b], SG[b]).wait()

    def s_start(b):
        pltpu.async_copy(M[b], acc.at[s].at[GB[b]], SS[b], add=True)

    def s_wait(b):
        pltpu.make_async_copy(M[b], acc.at[s].at[GB[b]], SS[b]).wait()

    zero16 = jnp.zeros((LANES,), jnp.float32)
    for r in range(G):
        for j in range(H // LANES):
            msgs0[r, pl.ds(j * LANES, LANES)] = zero16
    pltpu.sync_copy(msgs0.at[pl.ds(0, G), :], acc.at[s])

    def step(j, b, start_ahead, wait_prev):
        g_wait(j, b)
        _scale_rows(M[b], dstall, wgtab, GB[b], j * CB)
        pltpu.sync_copy(M[b], acc.at[s].at[GB[b]], add=True)
        if start_ahead:
            g_start(j + 2, (b + 2) % 3)

    g_start(0, 0)
    g_start(1, 1)
    step(0, 0, True, False)
    step(1, 1, True, True)
    step(2, 2, True, True)

    def body(t, carry):
        j0 = 3 * t
        step(j0, 0, True, True)
        step(j0 + 1, 1, True, True)
        step(j0 + 2, 2, True, True)
        return carry

    lax.fori_loop(1, CPT // 3 - 1, body, 0)
    j0 = CPT - 3
    step(j0, 0, True, True)
    step(j0 + 1, 1, False, True)
    step(j0 + 2, 2, False, True)

    @pl.when(w < NLEFT)
    def _():
        pltpu.sync_copy(src2d.at[NW * CPT + w], stail)
        pltpu.sync_copy(
            dst1d.at[pl.ds((NW * CPT + w) * CB, CB)],
            dstall.at[pl.ds(0, CB)])
        pltpu.async_copy(h.at[stail], msgs0, sem0).wait()
        _scale_rows(msgs0, dstall, wgtab, gbuf0, 0)
        pltpu.sync_copy(msgs0, acc.at[s].at[gbuf0], add=True)

    pltpu.sync_copy(acc.at[s], out.at[c, s])


def _tc1_body(p_ref, x_ref, batch_ref, wext_ref, h_ref, stats_ref, wg_ref):
    i = pl.program_id(0)
    agg = p_ref[0] + p_ref[1]
    deg = agg[:, 26:27]
    mean = agg / jnp.clip(deg, 1.0)
    inp = jnp.concatenate([mean, x_ref[...]], axis=1)
    hh = jnp.maximum(
        jnp.dot(inp, wext_ref[...], preferred_element_type=jnp.float32), 0.0)
    h_ref[...] = hh
    b = batch_ref[0, 0, :].reshape(R, 1)
    oh = (b == lax.broadcasted_iota(jnp.int32, (R, G), 1)).astype(jnp.float32)
    rhs = jnp.concatenate(
        [hh, jnp.ones((R, 1), jnp.float32),
         jnp.zeros((R, 128 - H - 1), jnp.float32)], axis=1)
    contrib = lax.dot_general(
        oh, rhs, (((0,), (0,)), ((), ())),
        preferred_element_type=jnp.float32)

    @pl.when(i == 0)
    def _():
        stats_ref[...] = jnp.zeros_like(stats_ref)

    stats_ref[...] += contrib

    wbits = lax.bitcast_convert_type(1.0 / jnp.clip(deg, 1.0), jnp.int32)
    packed = (wbits & jnp.int32(-64)) | b
    wg_ref[0, 0, :] = packed[:, 0]


def _tc2_body(pool_ref, stats_ref, w2l_ref, b2_ref, w2r_ref, wlin_ref,
              blin_ref, out_ref):
    agg2 = jnp.sum(pool_ref[...], axis=(0, 1))
    segh = stats_ref[:, :H]
    counts = stats_ref[:, H:H + 1]
    ps = (jnp.dot(agg2, w2l_ref[...], preferred_element_type=jnp.float32)
          + jnp.dot(segh, w2r_ref[...], preferred_element_type=jnp.float32)
          + counts * b2_ref[...])
    pooled = ps / jnp.clip(counts, 1.0)
    out_ref[...] = (
        jnp.dot(pooled, wlin_ref[...], preferred_element_type=jnp.float32)
        + blin_ref[...])


def kernel(x, edge_index, batch, W1_l, b1, W1_r, W2_l, b2, W2_r, Wlin, blin):
    src = edge_index[0].astype(jnp.int32)
    dst = edge_index[1].astype(jnp.int32)
    batch = batch.astype(jnp.int32)
    src2d = src.reshape(NCHUNKS, CB)
    dst2d = dst.reshape(NCHUNKS, CB)

    xpad = jnp.pad(
        jnp.concatenate([x, jnp.ones((N, 1), jnp.float32)], axis=1),
        ((0, NPAD - N), (0, FP - F_IN - 1)))

    mesh = plsc.VectorSubcoreMesh(core_axis_name="c", subcore_axis_name="s")
    sc_params = pltpu.CompilerParams(use_tc_tiling_on_sc=False,
                                     needs_layout_passes=False)

    l1 = pl.kernel(
        _l1_body,
        out_type=jax.ShapeDtypeStruct((NC, NPAD, FP), jnp.float32),
        mesh=mesh,
        compiler_params=sc_params,
        scratch_types=[
            pltpu.VMEM_SHARED((NPAD, FP), jnp.float32),
            pltpu.VMEM((ZROWS, FP), jnp.float32),
            pltpu.VMEM((IB, CB), jnp.int32),
            pltpu.VMEM((IB, CB), jnp.int32),
            pltpu.VMEM((IB, CB), jnp.int32),
            pltpu.VMEM((IB, CB), jnp.int32),
            pltpu.VMEM((CB,), jnp.int32),
            pltpu.VMEM((CB,), jnp.int32),
            pltpu.VMEM((CB, FP), jnp.float32),
            pltpu.VMEM((CB, FP), jnp.float32),
            pltpu.VMEM((CB, FP), jnp.float32),
            pltpu.SemaphoreType.DMA,
            pltpu.SemaphoreType.DMA,
            pltpu.SemaphoreType.DMA,
            pltpu.SemaphoreType.DMA,
            pltpu.SemaphoreType.DMA,
            pltpu.SemaphoreType.DMA,
            pltpu.SemaphoreType.DMA,
            pltpu.SemaphoreType.DMA,
        ],
    )
    partials = l1(xpad, src2d, dst2d)

    wext = jnp.zeros((2 * FP, H), jnp.float32)
    wext = wext.at[:F_IN, :].set(W1_l)
    wext = wext.at[FP:FP + F_IN, :].set(W1_r)
    wext = wext.at[FP + F_IN, :].set(b1)

    batch_pad = jnp.pad(batch, (0, NPAD - N), constant_values=G)
    batch3d = batch_pad.reshape(NPAD // R, 1, R)

    h, stats, wg3d = pl.pallas_call(
        _tc1_body,
        grid=(NPAD // R,),
        in_specs=[
            pl.BlockSpec((NC, R, FP), lambda i: (0, i, 0)),
            pl.BlockSpec((R, FP), lambda i: (i, 0)),
            pl.BlockSpec((1, 1, R), lambda i: (i, 0, 0)),
            pl.BlockSpec((2 * FP, H), lambda i: (0, 0)),
        ],
        out_specs=[
            pl.BlockSpec((R, H), lambda i: (i, 0)),
            pl.BlockSpec((G, 128), lambda i: (0, 0)),
            pl.BlockSpec((1, 1, R), lambda i: (i, 0, 0)),
        ],
        out_shape=[
            jax.ShapeDtypeStruct((NPAD, H), jnp.float32),
            jax.ShapeDtypeStruct((G, 128), jnp.float32),
            jax.ShapeDtypeStruct((NPAD // R, 1, R), jnp.int32),
        ],
    )(partials, xpad, batch3d, wext)
    wg = wg3d.reshape(NPAD)

    l2 = pl.kernel(
        _l2_body,
        out_type=jax.ShapeDtypeStruct((NC, NS, G, H), jnp.float32),
        mesh=mesh,
        compiler_params=sc_params,
        scratch_types=[
            pltpu.VMEM_SHARED((NS, G, H), jnp.float32),
            pltpu.VMEM((NPAD,), jnp.int32),
            pltpu.VMEM((CPT, CB), jnp.int32),
            pltpu.VMEM((CPT * CB,), jnp.int32),
            pltpu.VMEM((CB,), jnp.int32),
            pltpu.VMEM((CB,), jnp.int32),
            pltpu.VMEM((CB,), jnp.int32),
            pltpu.VMEM((CB,), jnp.int32),
            pltpu.VMEM((CB, H), jnp.float32),
            pltpu.VMEM((CB, H), jnp.float32),
            pltpu.VMEM((CB, H), jnp.float32),
            pltpu.SemaphoreType.DMA,
            pltpu.SemaphoreType.DMA,
            pltpu.SemaphoreType.DMA,
            pltpu.SemaphoreType.DMA,
            pltpu.SemaphoreType.DMA,
            pltpu.SemaphoreType.DMA,
        ],
    )
    pool = l2(h, src2d, dst, wg)

    out = pl.pallas_call(
        _tc2_body,
        out_shape=jax.ShapeDtypeStruct((G, F_IN), jnp.float32),
    )(pool, stats, W2_l, b2.reshape(1, H), W2_r, Wlin,
      blin.reshape(1, F_IN))
    return out


# speedup vs baseline: 1.0277x; 1.0277x over previous
import jax
import jax.numpy as jnp
from jax import lax
from jax.experimental import pallas as pl
from jax.experimental.pallas import tpu as pltpu
from jax.experimental.pallas import tpu_sc as plsc

N = 50000
NPAD = 50048
E = 800000
F_IN = 26
FP = 32
H = 64
G = 64
NC = 2
NS = 16
NW = NC * NS
CB = 128
NCHUNKS = E // CB
CPT = NCHUNKS // NW
NLEFT = NCHUNKS - CPT * NW
IB = 13
NBLK = CPT // IB
ROWS_PER_TILE = NPAD // NS
ZROWS = 184
R = 3128
LANES = 16


def _l1_body(xpad, src2d, dst2d, out, acc, zbuf, srcl0, dstl0, srcl1, dstl1,
             stail, dtail, msgs0, msgs1, msgs2, sem0, sem1, sem2,
             ssem0, ssem1, ssem2, isem, zsem):
    M = (msgs0, msgs1, msgs2)
    SG = (sem0, sem1, sem2)
    SS = (ssem0, ssem1, ssem2)
    c = lax.axis_index("c")
    s = lax.axis_index("s")
    w = c * NS + s

    def i_start(blk, sbuf, dbuf):
        cb0 = w * CPT + blk * IB
        pltpu.async_copy(src2d.at[pl.ds(cb0, IB), :], sbuf, isem)
        pltpu.async_copy(dst2d.at[pl.ds(cb0, IB), :], dbuf, isem)

    def i_wait(blk, sbuf, dbuf):
        cb0 = w * CPT + blk * IB
        pltpu.make_async_copy(src2d.at[pl.ds(cb0, IB), :], sbuf, isem).wait()
        pltpu.make_async_copy(dst2d.at[pl.ds(cb0, IB), :], dbuf, isem).wait()

    i_start(0, srcl0, dstl0)

    zero16 = jnp.zeros((LANES,), jnp.float32)
    for r in range(ZROWS):
        for j in range(FP // LANES):
            zbuf[r, pl.ds(j * LANES, LANES)] = zero16
    nz = ROWS_PER_TILE // ZROWS
    for i in range(nz):
        pltpu.async_copy(
            zbuf, acc.at[pl.ds(s * ROWS_PER_TILE + i * ZROWS, ZROWS), :],
            zsem)
    for i in range(nz):
        pltpu.make_async_copy(
            zbuf, acc.at[pl.ds(s * ROWS_PER_TILE + i * ZROWS, ZROWS), :],
            zsem).wait()
    plsc.subcore_barrier()

    def g_start(srcl, j, b):
        pltpu.async_copy(xpad.at[srcl.at[j]], M[b], SG[b])

    def g_wait(srcl, j, b):
        pltpu.make_async_copy(xpad.at[srcl.at[j]], M[b], SG[b]).wait()

    def s_start(dstl, j, b):
        pltpu.async_copy(M[b], acc.at[dstl.at[j]], SS[b], add=True)

    def s_wait(dstl, j, b):
        pltpu.make_async_copy(M[b], acc.at[dstl.at[j]], SS[b]).wait()

    def process(srcl, dstl):
        g_start(srcl, 0, 0)

        def body(t, carry2):
            c0 = 2 * t
            c1 = 2 * t + 1
            g_start(srcl, c1, 1)
            g_wait(srcl, c0, 0)
            pltpu.sync_copy(M[0], acc.at[dstl.at[c0]], add=True)
            g_start(srcl, c0 + 2, 0)
            g_wait(srcl, c1, 1)
            pltpu.sync_copy(M[1], acc.at[dstl.at[c1]], add=True)
            return carry2

        lax.fori_loop(0, IB // 2, body, 0)
        g_wait(srcl, IB - 1, 0)
        pltpu.sync_copy(M[0], acc.at[dstl.at[IB - 1]], add=True)

    def blk_body(t, carry):
        b0 = 2 * t
        b1 = 2 * t + 1
        i_start(b1, srcl1, dstl1)
        i_wait(b0, srcl0, dstl0)
        process(srcl0, dstl0)

        @pl.when(b0 + 2 < NBLK)
        def _():
            i_start(b0 + 2, srcl0, dstl0)

        i_wait(b1, srcl1, dstl1)
        process(srcl1, dstl1)
        return carry

    lax.fori_loop(0, NBLK // 2, blk_body, 0)
    i_wait(NBLK - 1, srcl0, dstl0)
    process(srcl0, dstl0)

    @pl.when(w < NLEFT)
    def _():
        pltpu.sync_copy(src2d.at[NW * CPT + w], stail)
        pltpu.sync_copy(dst2d.at[NW * CPT + w], dtail)
        pltpu.async_copy(xpad.at[stail], msgs1, sem1).wait()
        pltpu.sync_copy(msgs1, acc.at[dtail], add=True)

    plsc.subcore_barrier()
    pltpu.sync_copy(
        acc.at[pl.ds(s * ROWS_PER_TILE, ROWS_PER_TILE), :],
        out.at[c, pl.ds(s * ROWS_PER_TILE, ROWS_PER_TILE), :])


def _scale_rows(msgs, dstall, wgtab, gbuf, base):
    for k in range(CB // LANES):
        dv = dstall[pl.ds(base + k * LANES, LANES)]
        pk = plsc.load_gather(wgtab, [dv])
        gv = pk & jnp.int32(63)
        wv = plsc.bitcast(pk & jnp.int32(-64), jnp.float32)
        gbuf[pl.ds(k * LANES, LANES)] = gv
        for e in range(LANES):
            ws = jnp.broadcast_to(wv[e], (LANES,))
            row = k * LANES + e
            for j in range(H // LANES):
                sl = pl.ds(j * LANES, LANES)
                msgs[row, sl] = msgs[row, sl] * ws


def _l2_body(h, src2d, dst1d, wg, out, acc, wgtab, srcl, dstall, stail,
             gbuf0, gbuf1, gbuf2, msgs0, msgs1, msgs2,
             sem0, sem1, sem2, ssem0, ssem1, ssem2):
    c = lax.axis_index("c")
    s = lax.axis_index("s")
    w = c * NS + s
    M = (msgs0, msgs1, msgs2)
    GB = (gbuf0, gbuf1, gbuf2)
    SG = (sem0, sem1, sem2)
    SS = (ssem0, ssem1, ssem2)

    pltpu.sync_copy(src2d.at[pl.ds(w * CPT, CPT), :], srcl)
    pltpu.sync_copy(dst1d.at[pl.ds(w * CPT * CB, CPT * CB)], dstall)
    pltpu.sync_copy(wg, wgtab)

    def g_start(j, b):
        pltpu.async_copy(h.at[srcl.at[j]], M[b], SG[b])

    def g_wait(j, b):
        pltpu.make_async_copy(h.at[srcl.at[j]], M[b], SG[b]).wait()

    def s_start(b):
        pltpu.async_copy(M[b], acc.at[s].at[GB[b]], SS[b], add=True)

    def s_wait(b):
        pltpu.make_async_copy(M[b], acc.at[s].at[GB[b]], SS[b]).wait()

    zero16 = jnp.zeros((LANES,), jnp.float32)
    for r in range(G):
        for j in range(H // LANES):
            msgs0[r, pl.ds(j * LANES, LANES)] = zero16
    pltpu.sync_copy(msgs0.at[pl.ds(0, G), :], acc.at[s])

    def step(j, b, start_ahead, wait_prev):
        g_wait(j, b)
        _scale_rows(M[b], dstall, wgtab, GB[b], j * CB)
        s_start(b)
        if wait_prev:
            s_wait((b + 2) % 3)
        if start_ahead:
            g_start(j + 2, (b + 2) % 3)

    g_start(0, 0)
    g_start(1, 1)
    step(0, 0, True, False)
    step(1, 1, True, True)
    step(2, 2, True, True)

    def body(t, carry):
        j0 = 3 * t
        step(j0, 0, True, True)
        step(j0 + 1, 1, True, True)
        step(j0 + 2, 2, True, True)
        return carry

    lax.fori_loop(1, CPT // 3 - 1, body, 0)
    j0 = CPT - 3
    step(j0, 0, True, True)
    step(j0 + 1, 1, False, True)
    step(j0 + 2, 2, False, True)
    s_wait(2)

    @pl.when(w < NLEFT)
    def _():
        pltpu.sync_copy(src2d.at[NW * CPT + w], stail)
        pltpu.sync_copy(
            dst1d.at[pl.ds((NW * CPT + w) * CB, CB)],
            dstall.at[pl.ds(0, CB)])
        pltpu.async_copy(h.at[stail], msgs0, sem0).wait()
        _scale_rows(msgs0, dstall, wgtab, gbuf0, 0)
        pltpu.sync_copy(msgs0, acc.at[s].at[gbuf0], add=True)

    pltpu.sync_copy(acc.at[s], out.at[c, s])


def _tc1_body(p_ref, x_ref, batch_ref, wext_ref, h_ref, stats_ref, wg_ref):
    i = pl.program_id(0)
    agg = p_ref[0] + p_ref[1]
    deg = agg[:, 26:27]
    mean = agg / jnp.clip(deg, 1.0)
    inp = jnp.concatenate([mean, x_ref[...]], axis=1)
    hh = jnp.maximum(
        jnp.dot(inp, wext_ref[...], preferred_element_type=jnp.float32), 0.0)
    h_ref[...] = hh
    b = batch_ref[0, 0, :].reshape(R, 1)
    oh = (b == lax.broadcasted_iota(jnp.int32, (R, G), 1)).astype(jnp.float32)
    rhs = jnp.concatenate(
        [hh, jnp.ones((R, 1), jnp.float32),
         jnp.zeros((R, 128 - H - 1), jnp.float32)], axis=1)
    contrib = lax.dot_general(
        oh, rhs, (((0,), (0,)), ((), ())),
        preferred_element_type=jnp.float32)

    @pl.when(i == 0)
    def _():
        stats_ref[...] = jnp.zeros_like(stats_ref)

    stats_ref[...] += contrib

    wbits = lax.bitcast_convert_type(1.0 / jnp.clip(deg, 1.0), jnp.int32)
    packed = (wbits & jnp.int32(-64)) | b
    wg_ref[0, 0, :] = packed[:, 0]


def _tc2_body(pool_ref, stats_ref, w2l_ref, b2_ref, w2r_ref, wlin_ref,
              blin_ref, out_ref):
    agg2 = jnp.sum(pool_ref[...], axis=(0, 1))
    segh = stats_ref[:, :H]
    counts = stats_ref[:, H:H + 1]
    ps = (jnp.dot(agg2, w2l_ref[...], preferred_element_type=jnp.float32)
          + jnp.dot(segh, w2r_ref[...], preferred_element_type=jnp.float32)
          + counts * b2_ref[...])
    pooled = ps / jnp.clip(counts, 1.0)
    out_ref[...] = (
        jnp.dot(pooled, wlin_ref[...], preferred_element_type=jnp.float32)
        + blin_ref[...])


def kernel(x, edge_index, batch, W1_l, b1, W1_r, W2_l, b2, W2_r, Wlin, blin):
    src = edge_index[0].astype(jnp.int32)
    dst = edge_index[1].astype(jnp.int32)
    batch = batch.astype(jnp.int32)
    src2d = src.reshape(NCHUNKS, CB)
    dst2d = dst.reshape(NCHUNKS, CB)

    xpad = jnp.pad(
        jnp.concatenate([x, jnp.ones((N, 1), jnp.float32)], axis=1),
        ((0, NPAD - N), (0, FP - F_IN - 1)))

    mesh = plsc.VectorSubcoreMesh(core_axis_name="c", subcore_axis_name="s")
    sc_params = pltpu.CompilerParams(use_tc_tiling_on_sc=False,
                                     needs_layout_passes=False)

    l1 = pl.kernel(
        _l1_body,
        out_type=jax.ShapeDtypeStruct((NC, NPAD, FP), jnp.float32),
        mesh=mesh,
        compiler_params=sc_params,
        scratch_types=[
            pltpu.VMEM_SHARED((NPAD, FP), jnp.float32),
            pltpu.VMEM((ZROWS, FP), jnp.float32),
            pltpu.VMEM((IB, CB), jnp.int32),
            pltpu.VMEM((IB, CB), jnp.int32),
            pltpu.VMEM((IB, CB), jnp.int32),
            pltpu.VMEM((IB, CB), jnp.int32),
            pltpu.VMEM((CB,), jnp.int32),
            pltpu.VMEM((CB,), jnp.int32),
            pltpu.VMEM((CB, FP), jnp.float32),
            pltpu.VMEM((CB, FP), jnp.float32),
            pltpu.VMEM((CB, FP), jnp.float32),
            pltpu.SemaphoreType.DMA,
            pltpu.SemaphoreType.DMA,
            pltpu.SemaphoreType.DMA,
            pltpu.SemaphoreType.DMA,
            pltpu.SemaphoreType.DMA,
            pltpu.SemaphoreType.DMA,
            pltpu.SemaphoreType.DMA,
            pltpu.SemaphoreType.DMA,
        ],
    )
    partials = l1(xpad, src2d, dst2d)

    wext = jnp.zeros((2 * FP, H), jnp.float32)
    wext = wext.at[:F_IN, :].set(W1_l)
    wext = wext.at[FP:FP + F_IN, :].set(W1_r)
    wext = wext.at[FP + F_IN, :].set(b1)

    batch_pad = jnp.pad(batch, (0, NPAD - N), constant_values=G)
    batch3d = batch_pad.reshape(NPAD // R, 1, R)

    h, stats, wg3d = pl.pallas_call(
        _tc1_body,
        grid=(NPAD // R,),
        in_specs=[
            pl.BlockSpec((NC, R, FP), lambda i: (0, i, 0)),
            pl.BlockSpec((R, FP), lambda i: (i, 0)),
            pl.BlockSpec((1, 1, R), lambda i: (i, 0, 0)),
            pl.BlockSpec((2 * FP, H), lambda i: (0, 0)),
        ],
        out_specs=[
            pl.BlockSpec((R, H), lambda i: (i, 0)),
            pl.BlockSpec((G, 128), lambda i: (0, 0)),
            pl.BlockSpec((1, 1, R), lambda i: (i, 0, 0)),
        ],
        out_shape=[
            jax.ShapeDtypeStruct((NPAD, H), jnp.float32),
            jax.ShapeDtypeStruct((G, 128), jnp.float32),
            jax.ShapeDtypeStruct((NPAD // R, 1, R), jnp.int32),
        ],
    )(partials, xpad, batch3d, wext)
    wg = wg3d.reshape(NPAD)

    l2 = pl.kernel(
        _l2_body,
        out_type=jax.ShapeDtypeStruct((NC, NS, G, H), jnp.float32),
        mesh=mesh,
        compiler_params=sc_params,
        scratch_types=[
            pltpu.VMEM_SHARED((NS, G, H), jnp.float32),
            pltpu.VMEM((NPAD,), jnp.int32),
            pltpu.VMEM((CPT, CB), jnp.int32),
            pltpu.VMEM((CPT * CB,), jnp.int32),
            pltpu.VMEM((CB,), jnp.int32),
            pltpu.VMEM((CB,), jnp.int32),
            pltpu.VMEM((CB,), jnp.int32),
            pltpu.VMEM((CB,), jnp.int32),
            pltpu.VMEM((CB, H), jnp.float32),
            pltpu.VMEM((CB, H), jnp.float32),
            pltpu.VMEM((CB, H), jnp.float32),
            pltpu.SemaphoreType.DMA,
            pltpu.SemaphoreType.DMA,
            pltpu.SemaphoreType.DMA,
            pltpu.SemaphoreType.DMA,
            pltpu.SemaphoreType.DMA,
            pltpu.SemaphoreType.DMA,
        ],
    )
    pool = l2(h, src2d, dst, wg)

    out = pl.pallas_call(
        _tc2_body,
        out_shape=jax.ShapeDtypeStruct((G, F_IN), jnp.float32),
    )(pool, stats, W2_l, b2.reshape(1, H), W2_r, Wlin,
      blin.reshape(1, F_IN))
    return out


# speedup vs baseline: 1.0838x; 1.0546x over previous
import jax
import jax.numpy as jnp
from jax import lax
from jax.experimental import pallas as pl
from jax.experimental.pallas import tpu as pltpu
from jax.experimental.pallas import tpu_sc as plsc

N = 50000
NPAD = 50048
E = 800000
F_IN = 26
FP = 32
H = 64
G = 64
NC = 2
NS = 16
NW = NC * NS
CB = 128
NCHUNKS = E // CB
CPT = NCHUNKS // NW
NLEFT = NCHUNKS - CPT * NW
IB = 13
NBLK = CPT // IB
ROWS_PER_TILE = NPAD // NS
ZROWS = 184
R = 3128
LANES = 16


def _l1_body(xpad, src2d, dst2d, out, acc, zbuf, srcl0, dstl0, srcl1, dstl1,
             stail, dtail, msgs0, msgs1, msgs2, sem0, sem1, sem2,
             ssem0, ssem1, ssem2, isem, zsem):
    M = (msgs0, msgs1, msgs2)
    SG = (sem0, sem1, sem2)
    SS = (ssem0, ssem1, ssem2)
    c = lax.axis_index("c")
    s = lax.axis_index("s")
    w = c * NS + s

    def i_start(blk, sbuf, dbuf):
        cb0 = w * CPT + blk * IB
        pltpu.async_copy(src2d.at[pl.ds(cb0, IB), :], sbuf, isem)
        pltpu.async_copy(dst2d.at[pl.ds(cb0, IB), :], dbuf, isem)

    def i_wait(blk, sbuf, dbuf):
        cb0 = w * CPT + blk * IB
        pltpu.make_async_copy(src2d.at[pl.ds(cb0, IB), :], sbuf, isem).wait()
        pltpu.make_async_copy(dst2d.at[pl.ds(cb0, IB), :], dbuf, isem).wait()

    i_start(0, srcl0, dstl0)

    zero16 = jnp.zeros((LANES,), jnp.float32)
    for r in range(ZROWS):
        for j in range(FP // LANES):
            zbuf[r, pl.ds(j * LANES, LANES)] = zero16
    nz = ROWS_PER_TILE // ZROWS
    for i in range(nz):
        pltpu.async_copy(
            zbuf, acc.at[pl.ds(s * ROWS_PER_TILE + i * ZROWS, ZROWS), :],
            zsem)
    for i in range(nz):
        pltpu.make_async_copy(
            zbuf, acc.at[pl.ds(s * ROWS_PER_TILE + i * ZROWS, ZROWS), :],
            zsem).wait()
    plsc.subcore_barrier()

    def g_start(srcl, j, b):
        pltpu.async_copy(xpad.at[srcl.at[j]], M[b], SG[b])

    def g_wait(srcl, j, b):
        pltpu.make_async_copy(xpad.at[srcl.at[j]], M[b], SG[b]).wait()

    def s_start(dstl, j, b):
        pltpu.async_copy(M[b], acc.at[dstl.at[j]], SS[b], add=True)

    def s_wait(dstl, j, b):
        pltpu.make_async_copy(M[b], acc.at[dstl.at[j]], SS[b]).wait()

    def process(srcl, dstl):
        g_start(srcl, 0, 0)

        def body(t, carry2):
            c0 = 2 * t
            c1 = 2 * t + 1
            g_start(srcl, c1, 1)
            g_wait(srcl, c0, 0)
            pltpu.sync_copy(M[0], acc.at[dstl.at[c0]], add=True)
            g_start(srcl, c0 + 2, 0)
            g_wait(srcl, c1, 1)
            pltpu.sync_copy(M[1], acc.at[dstl.at[c1]], add=True)
            return carry2

        lax.fori_loop(0, IB // 2, body, 0)
        g_wait(srcl, IB - 1, 0)
        pltpu.sync_copy(M[0], acc.at[dstl.at[IB - 1]], add=True)

    def blk_body(t, carry):
        b0 = 2 * t
        b1 = 2 * t + 1
        i_start(b1, srcl1, dstl1)
        i_wait(b0, srcl0, dstl0)
        process(srcl0, dstl0)

        @pl.when(b0 + 2 < NBLK)
        def _():
            i_start(b0 + 2, srcl0, dstl0)

        i_wait(b1, srcl1, dstl1)
        process(srcl1, dstl1)
        return carry

    lax.fori_loop(0, NBLK // 2, blk_body, 0)
    i_wait(NBLK - 1, srcl0, dstl0)
    process(srcl0, dstl0)

    @pl.when(w < NLEFT)
    def _():
        pltpu.sync_copy(src2d.at[NW * CPT + w], stail)
        pltpu.sync_copy(dst2d.at[NW * CPT + w], dtail)
        pltpu.async_copy(xpad.at[stail], msgs1, sem1).wait()
        pltpu.sync_copy(msgs1, acc.at[dtail], add=True)

    plsc.subcore_barrier()
    pltpu.sync_copy(
        acc.at[pl.ds(s * ROWS_PER_TILE, ROWS_PER_TILE), :],
        out.at[c, pl.ds(s * ROWS_PER_TILE, ROWS_PER_TILE), :])


def _scale_rows(msgs, dstall, wgtab, gbuf, base):
    for k in range(CB // LANES):
        dv = dstall[pl.ds(base + k * LANES, LANES)]
        pk = plsc.load_gather(wgtab, [dv])
        gv = pk & jnp.int32(63)
        wv = plsc.bitcast(pk & jnp.int32(-64), jnp.float32)
        gbuf[pl.ds(k * LANES, LANES)] = gv
        for e in range(LANES):
            ws = jnp.broadcast_to(wv[e], (LANES,))
            row = k * LANES + e
            for j in range(H // LANES):
                sl = pl.ds(j * LANES, LANES)
                msgs[row, sl] = msgs[row, sl] * ws


def _l2_body(h, src2d, dst1d, wg, out, acc, wgtab, srcl, dstall, stail,
             gbuf0, gbuf1, gbuf2, msgs0, msgs1, msgs2,
             sem0, sem1, sem2, ssem0, ssem1, ssem2):
    c = lax.axis_index("c")
    s = lax.axis_index("s")
    w = c * NS + s
    M = (msgs0, msgs1, msgs2)
    GB = (gbuf0, gbuf1, gbuf2)
    SG = (sem0, sem1, sem2)
    SS = (ssem0, ssem1, ssem2)

    pltpu.sync_copy(src2d.at[pl.ds(w * CPT, CPT), :], srcl)
    pltpu.sync_copy(dst1d.at[pl.ds(w * CPT * CB, CPT * CB)], dstall)
    pltpu.sync_copy(wg, wgtab)

    def g_start(j, b):
        pltpu.async_copy(h.at[srcl.at[j]], M[b], SG[b])

    def g_wait(j, b):
        pltpu.make_async_copy(h.at[srcl.at[j]], M[b], SG[b]).wait()

    def s_start(b):
        pltpu.async_copy(M[b], acc.at[s].at[GB[b]], SS[b], add=True)

    def s_wait(b):
        pltpu.make_async_copy(M[b], acc.at[s].at[GB[b]], SS[b]).wait()

    zero16 = jnp.zeros((LANES,), jnp.float32)
    for r in range(G):
        for j in range(H // LANES):
            msgs0[r, pl.ds(j * LANES, LANES)] = zero16
    pltpu.sync_copy(msgs0.at[pl.ds(0, G), :], acc.at[s])

    def consume(j, b):
        g_wait(j, b)
        _scale_rows(M[b], dstall, wgtab, GB[b], j * CB)
        pltpu.sync_copy(M[b], acc.at[s].at[GB[b]], add=True)

    g_start(0, 0)

    def body(t, carry):
        c0 = 2 * t
        c1 = 2 * t + 1
        g_start(c1, 1)
        consume(c0, 0)

        @pl.when(c0 + 2 < CPT)
        def _():
            g_start(c0 + 2, 0)

        consume(c1, 1)
        return carry

    lax.fori_loop(0, CPT // 2, body, 0)
    consume(CPT - 1, 0)

    @pl.when(w < NLEFT)
    def _():
        pltpu.sync_copy(src2d.at[NW * CPT + w], stail)
        pltpu.sync_copy(
            dst1d.at[pl.ds((NW * CPT + w) * CB, CB)],
            dstall.at[pl.ds(0, CB)])
        pltpu.async_copy(h.at[stail], msgs0, sem0).wait()
        _scale_rows(msgs0, dstall, wgtab, gbuf0, 0)
        pltpu.sync_copy(msgs0, acc.at[s].at[gbuf0], add=True)

    pltpu.sync_copy(acc.at[s], out.at[c, s])


def _tc1_body(p_ref, x_ref, batch_ref, wext_ref, h_ref, stats_ref, wg_ref):
    i = pl.program_id(0)
    agg = p_ref[0] + p_ref[1]
    deg = agg[:, 26:27]
    mean = agg / jnp.clip(deg, 1.0)
    inp = jnp.concatenate([mean, x_ref[...]], axis=1)
    hh = jnp.maximum(
        jnp.dot(inp, wext_ref[...], preferred_element_type=jnp.float32), 0.0)
    h_ref[...] = hh
    b = batch_ref[0, 0, :].reshape(R, 1)
    oh = (b == lax.broadcasted_iota(jnp.int32, (R, G), 1)).astype(jnp.float32)
    rhs = jnp.concatenate(
        [hh, jnp.ones((R, 1), jnp.float32),
         jnp.zeros((R, 128 - H - 1), jnp.float32)], axis=1)
    contrib = lax.dot_general(
        oh, rhs, (((0,), (0,)), ((), ())),
        preferred_element_type=jnp.float32)

    @pl.when(i == 0)
    def _():
        stats_ref[...] = jnp.zeros_like(stats_ref)

    stats_ref[...] += contrib

    wbits = lax.bitcast_convert_type(1.0 / jnp.clip(deg, 1.0), jnp.int32)
    packed = (wbits & jnp.int32(-64)) | b
    wg_ref[0, 0, :] = packed[:, 0]


def _tc2_body(pool_ref, stats_ref, w2l_ref, b2_ref, w2r_ref, wlin_ref,
              blin_ref, out_ref):
    agg2 = jnp.sum(pool_ref[...], axis=(0, 1))
    segh = stats_ref[:, :H]
    counts = stats_ref[:, H:H + 1]
    ps = (jnp.dot(agg2, w2l_ref[...], preferred_element_type=jnp.float32)
          + jnp.dot(segh, w2r_ref[...], preferred_element_type=jnp.float32)
          + counts * b2_ref[...])
    pooled = ps / jnp.clip(counts, 1.0)
    out_ref[...] = (
        jnp.dot(pooled, wlin_ref[...], preferred_element_type=jnp.float32)
        + blin_ref[...])


def kernel(x, edge_index, batch, W1_l, b1, W1_r, W2_l, b2, W2_r, Wlin, blin):
    src = edge_index[0].astype(jnp.int32)
    dst = edge_index[1].astype(jnp.int32)
    batch = batch.astype(jnp.int32)
    src2d = src.reshape(NCHUNKS, CB)
    dst2d = dst.reshape(NCHUNKS, CB)

    xpad = jnp.pad(
        jnp.concatenate([x, jnp.ones((N, 1), jnp.float32)], axis=1),
        ((0, NPAD - N), (0, FP - F_IN - 1)))

    mesh = plsc.VectorSubcoreMesh(core_axis_name="c", subcore_axis_name="s")
    sc_params = pltpu.CompilerParams(use_tc_tiling_on_sc=False,
                                     needs_layout_passes=False)

    l1 = pl.kernel(
        _l1_body,
        out_type=jax.ShapeDtypeStruct((NC, NPAD, FP), jnp.float32),
        mesh=mesh,
        compiler_params=sc_params,
        scratch_types=[
            pltpu.VMEM_SHARED((NPAD, FP), jnp.float32),
            pltpu.VMEM((ZROWS, FP), jnp.float32),
            pltpu.VMEM((IB, CB), jnp.int32),
            pltpu.VMEM((IB, CB), jnp.int32),
            pltpu.VMEM((IB, CB), jnp.int32),
            pltpu.VMEM((IB, CB), jnp.int32),
            pltpu.VMEM((CB,), jnp.int32),
            pltpu.VMEM((CB,), jnp.int32),
            pltpu.VMEM((CB, FP), jnp.float32),
            pltpu.VMEM((CB, FP), jnp.float32),
            pltpu.VMEM((CB, FP), jnp.float32),
            pltpu.SemaphoreType.DMA,
            pltpu.SemaphoreType.DMA,
            pltpu.SemaphoreType.DMA,
            pltpu.SemaphoreType.DMA,
            pltpu.SemaphoreType.DMA,
            pltpu.SemaphoreType.DMA,
            pltpu.SemaphoreType.DMA,
            pltpu.SemaphoreType.DMA,
        ],
    )
    partials = l1(xpad, src2d, dst2d)

    wext = jnp.zeros((2 * FP, H), jnp.float32)
    wext = wext.at[:F_IN, :].set(W1_l)
    wext = wext.at[FP:FP + F_IN, :].set(W1_r)
    wext = wext.at[FP + F_IN, :].set(b1)

    batch_pad = jnp.pad(batch, (0, NPAD - N), constant_values=G)
    batch3d = batch_pad.reshape(NPAD // R, 1, R)

    h, stats, wg3d = pl.pallas_call(
        _tc1_body,
        grid=(NPAD // R,),
        in_specs=[
            pl.BlockSpec((NC, R, FP), lambda i: (0, i, 0)),
            pl.BlockSpec((R, FP), lambda i: (i, 0)),
            pl.BlockSpec((1, 1, R), lambda i: (i, 0, 0)),
            pl.BlockSpec((2 * FP, H), lambda i: (0, 0)),
        ],
        out_specs=[
            pl.BlockSpec((R, H), lambda i: (i, 0)),
            pl.BlockSpec((G, 128), lambda i: (0, 0)),
            pl.BlockSpec((1, 1, R), lambda i: (i, 0, 0)),
        ],
        out_shape=[
            jax.ShapeDtypeStruct((NPAD, H), jnp.float32),
            jax.ShapeDtypeStruct((G, 128), jnp.float32),
            jax.ShapeDtypeStruct((NPAD // R, 1, R), jnp.int32),
        ],
    )(partials, xpad, batch3d, wext)
    wg = wg3d.reshape(NPAD)

    l2 = pl.kernel(
        _l2_body,
        out_type=jax.ShapeDtypeStruct((NC, NS, G, H), jnp.float32),
        mesh=mesh,
        compiler_params=sc_params,
        scratch_types=[
            pltpu.VMEM_SHARED((NS, G, H), jnp.float32),
            pltpu.VMEM((NPAD,), jnp.int32),
            pltpu.VMEM((CPT, CB), jnp.int32),
            pltpu.VMEM((CPT * CB,), jnp.int32),
            pltpu.VMEM((CB,), jnp.int32),
            pltpu.VMEM((CB,), jnp.int32),
            pltpu.VMEM((CB,), jnp.int32),
            pltpu.VMEM((CB,), jnp.int32),
            pltpu.VMEM((CB, H), jnp.float32),
            pltpu.VMEM((CB, H), jnp.float32),
            pltpu.VMEM((CB, H), jnp.float32),
            pltpu.SemaphoreType.DMA,
            pltpu.SemaphoreType.DMA,
            pltpu.SemaphoreType.DMA,
            pltpu.SemaphoreType.DMA,
            pltpu.SemaphoreType.DMA,
            pltpu.SemaphoreType.DMA,
        ],
    )
    pool = l2(h, src2d, dst, wg)

    out = pl.pallas_call(
        _tc2_body,
        out_shape=jax.ShapeDtypeStruct((G, F_IN), jnp.float32),
    )(pool, stats, W2_l, b2.reshape(1, H), W2_r, Wlin,
      blin.reshape(1, F_IN))
    return out
